# baseline (speedup 1.0000x reference)
import functools

import jax
import jax.numpy as jnp
from jax import lax
from jax.experimental import pallas as pl
from jax.experimental.pallas import tpu as pltpu
from jax.experimental.pallas import tpu_sc as plsc

_NC = 2
_NS = 16
_C2 = 128
_K = 3


def _pad_rows(n):
    g = _NS * 8
    return ((n + g - 1) // g) * g


def _pad_edges(a, fill, n_nodes):
    e = a.shape[0]
    nw = _NC * _NS
    assert e % nw == 0, e
    per = e // nw
    cpt = -(-per // (_C2 * _K)) * _K
    pad = cpt * _C2 - per
    a2 = a.reshape(nw, per)
    if pad:
        a2 = jnp.concatenate(
            [a2, jnp.full((nw, pad), fill, a.dtype)], axis=1)
    return a2.reshape(nw * cpt * _C2), cpt


def _sc_degree(dst2, cpt, n_nodes):
    n_pad = _pad_rows(n_nodes)
    rpt = n_pad // _NS

    mesh = plsc.VectorSubcoreMesh(core_axis_name="c", subcore_axis_name="s")

    @functools.partial(
        pl.kernel,
        out_type=jax.ShapeDtypeStruct((_NC * n_pad,), jnp.float32),
        mesh=mesh,
        scratch_types=[
            pltpu.VMEM((_C2,), jnp.int32),
            pltpu.VMEM((_C2,), jnp.float32),
            pltpu.VMEM((rpt,), jnp.float32),
            pltpu.VMEM_SHARED((n_pad,), jnp.float32),
            pltpu.SemaphoreType.DMA,
        ],
    )
    def deg_kernel(dst_hbm, ones_h, zeros_h, out_hbm, didx_v, ones_v,
                   stage_v, acc_sh, sem):
        cid = lax.axis_index("c")
        sid = lax.axis_index("s")
        wid = cid * _NS + sid
        pltpu.sync_copy(zeros_h.at[pl.ds(sid * rpt, rpt)], stage_v)
        pltpu.sync_copy(stage_v, acc_sh.at[pl.ds(sid * rpt, rpt)])
        pltpu.sync_copy(ones_h, ones_v)
        plsc.subcore_barrier()
        base0 = wid * cpt * _C2

        def body(j, carry):
            pltpu.sync_copy(dst_hbm.at[pl.ds(base0 + j * _C2, _C2)], didx_v)
            pltpu.sync_copy(ones_v, acc_sh.at[didx_v], add=True)
            return carry

        lax.fori_loop(0, cpt, body, 0)
        plsc.subcore_barrier()
        pltpu.sync_copy(acc_sh.at[pl.ds(sid * rpt, rpt)], stage_v)
        pltpu.sync_copy(stage_v,
                        out_hbm.at[pl.ds(cid * n_pad + sid * rpt, rpt)])

    ones = jnp.ones((_C2,), jnp.float32)
    zeros = jnp.zeros((n_pad,), jnp.float32)
    return deg_kernel(dst2, ones, zeros)


def _sc_propagate(table, src2, dst2, cpt):
    n_nodes, d = table.shape
    n_pad = _pad_rows(n_nodes)
    rpt = n_pad // _NS
    full = n_nodes // rpt
    rem = n_nodes - full * rpt
    assert rem % 8 == 0

    mesh = plsc.VectorSubcoreMesh(core_axis_name="c", subcore_axis_name="s")

    @functools.partial(
        pl.kernel,
        out_type=jax.ShapeDtypeStruct((_NC, n_pad, d), jnp.float32),
        mesh=mesh,
        scratch_types=[
            [pltpu.VMEM((_C2,), jnp.int32)] * _K,
            [pltpu.VMEM((_C2,), jnp.int32)] * _K,
            [pltpu.VMEM((_C2, d), jnp.float32)] * _K,
            pltpu.VMEM_SHARED((n_pad, d), jnp.float32),
            [pltpu.SemaphoreType.DMA] * _K,
            [pltpu.SemaphoreType.DMA] * _K,
            [pltpu.SemaphoreType.DMA] * _K,
        ],
    )
    def prop_kernel(table_hbm, src_hbm, dst_hbm, zeros_h, out_hbm,
                    sidx_v, didx_v, rows_v, acc_sh, semi, semg, sems):
        cid = lax.axis_index("c")
        sid = lax.axis_index("s")
        wid = cid * _NS + sid
        pltpu.sync_copy(zeros_h.at[pl.ds(sid * rpt, rpt)],
                        acc_sh.at[pl.ds(sid * rpt, rpt)])
        plsc.subcore_barrier()
        base0 = wid * cpt * _C2

        def body(i, carry):
            for k in range(_K):
                base = base0 + (i * _K + k) * _C2

                @pl.when(i > 0)
                def _(k=k):
                    pltpu.make_async_copy(
                        rows_v[k], acc_sh.at[didx_v[k]], sems[k]).wait()

                pltpu.async_copy(src_hbm.at[pl.ds(base, _C2)],
                                 sidx_v[k], semi[k])
                pltpu.async_copy(dst_hbm.at[pl.ds(base, _C2)],
                                 didx_v[k], semi[k])
            gd = []
            for k in range(_K):
                pltpu.make_async_copy(src_hbm.at[pl.ds(base0, _C2)],
                                      sidx_v[k], semi[k]).wait()
                pltpu.make_async_copy(dst_hbm.at[pl.ds(base0, _C2)],
                                      didx_v[k], semi[k]).wait()
                gd.append(pltpu.async_copy(table_hbm.at[sidx_v[k]],
                                           rows_v[k], semg[k]))
            for k in range(_K):
                gd[k].wait()
                pltpu.async_copy(rows_v[k], acc_sh.at[didx_v[k]],
                                 sems[k], add=True)
            return carry

        lax.fori_loop(0, cpt // _K, body, 0)
        for k in range(_K):
            pltpu.make_async_copy(rows_v[k], acc_sh.at[didx_v[k]],
                                  sems[k]).wait()
        plsc.subcore_barrier()
        pltpu.sync_copy(acc_sh.at[pl.ds(sid * rpt, rpt)],
                        out_hbm.at[cid, pl.ds(sid * rpt, rpt)])

    zeros = jnp.zeros((n_pad, d), jnp.float32)
    return prop_kernel(table, src2, dst2, zeros)


_ROWS = 1000


def _tc_pre(x, deg0, deg1, w):
    n, d = x.shape
    dout = w.shape[1]
    assert n % _ROWS == 0

    def body(x_ref, d0_ref, d1_ref, w_ref, o_ref):
        dinv = lax.rsqrt(d0_ref[...] + d1_ref[...] + 1.0)
        o_ref[...] = jnp.dot(x_ref[...] * dinv, w_ref[...],
                             preferred_element_type=jnp.float32)

    return pl.pallas_call(
        body,
        grid=(n // _ROWS,),
        in_specs=[
            pl.BlockSpec((_ROWS, d), lambda i: (i, 0)),
            pl.BlockSpec((_ROWS, 1), lambda i: (i, 0)),
            pl.BlockSpec((_ROWS, 1), lambda i: (i, 0)),
            pl.BlockSpec((d, dout), lambda i: (0, 0)),
        ],
        out_specs=pl.BlockSpec((_ROWS, dout), lambda i: (i, 0)),
        out_shape=jax.ShapeDtypeStruct((n, dout), jnp.float32),
    )(x, deg0, deg1, w)


def _tc_mid(p0, p1, z, deg0, deg1, b, w):
    n, d = z.shape
    dout = w.shape[1]

    def body(p0_ref, p1_ref, z_ref, d0_ref, d1_ref, b_ref, w_ref, o_ref):
        dinv = lax.rsqrt(d0_ref[...] + d1_ref[...] + 1.0)
        pre = (p0_ref[...] + p1_ref[...] + z_ref[...]) * dinv + b_ref[...]
        h = jnp.maximum(pre, 0.0) * dinv
        o_ref[...] = jnp.dot(h, w_ref[...], preferred_element_type=jnp.float32)

    return pl.pallas_call(
        body,
        grid=(n // _ROWS,),
        in_specs=[
            pl.BlockSpec((_ROWS, d), lambda i: (i, 0)),
            pl.BlockSpec((_ROWS, d), lambda i: (i, 0)),
            pl.BlockSpec((_ROWS, d), lambda i: (i, 0)),
            pl.BlockSpec((_ROWS, 1), lambda i: (i, 0)),
            pl.BlockSpec((_ROWS, 1), lambda i: (i, 0)),
            pl.BlockSpec((1, d), lambda i: (0, 0)),
            pl.BlockSpec((d, dout), lambda i: (0, 0)),
        ],
        out_specs=pl.BlockSpec((_ROWS, dout), lambda i: (i, 0)),
        out_shape=jax.ShapeDtypeStruct((n, dout), jnp.float32),
    )(p0, p1, z, deg0, deg1, b, w)


def _tc_post(p0, p1, z, deg0, deg1, b, wh, bh):
    n, d = z.shape
    dout = wh.shape[1]

    def body(p0_ref, p1_ref, z_ref, d0_ref, d1_ref, b_ref, wh_ref, bh_ref,
             o_ref):
        dinv = lax.rsqrt(d0_ref[...] + d1_ref[...] + 1.0)
        pre = (p0_ref[...] + p1_ref[...] + z_ref[...]) * dinv + b_ref[...]
        h = jnp.maximum(pre, 0.0)
        o_ref[...] = jnp.dot(h, wh_ref[...],
                             preferred_element_type=jnp.float32) + bh_ref[...]

    return pl.pallas_call(
        body,
        grid=(n // _ROWS,),
        in_specs=[
            pl.BlockSpec((_ROWS, d), lambda i: (i, 0)),
            pl.BlockSpec((_ROWS, d), lambda i: (i, 0)),
            pl.BlockSpec((_ROWS, d), lambda i: (i, 0)),
            pl.BlockSpec((_ROWS, 1), lambda i: (i, 0)),
            pl.BlockSpec((_ROWS, 1), lambda i: (i, 0)),
            pl.BlockSpec((1, d), lambda i: (0, 0)),
            pl.BlockSpec((d, dout), lambda i: (0, 0)),
            pl.BlockSpec((1, dout), lambda i: (0, 0)),
        ],
        out_specs=pl.BlockSpec((_ROWS, dout), lambda i: (i, 0)),
        out_shape=jax.ShapeDtypeStruct((n, dout), jnp.float32),
    )(p0, p1, z, deg0, deg1, b, wh, bh)


def kernel(x, edge_index, W1, b1, W2, b2, Wh, bh):
    n, d = x.shape
    src = edge_index[0].astype(jnp.int32)
    dst = edge_index[1].astype(jnp.int32)
    src2, cpt = _pad_edges(src, 0, n)
    dst2, _ = _pad_edges(dst, n, n)

    n_pad = _pad_rows(n)
    deg2 = _sc_degree(dst2, cpt, n).reshape(_NC, n_pad, 1)
    deg_a, deg_b = deg2[0], deg2[1]

    z1 = _tc_pre(x, deg_a, deg_b, W1)
    pp = _sc_propagate(z1, src2, dst2, cpt)
    z2 = _tc_mid(pp[0], pp[1], z1, deg_a, deg_b, b1.reshape(1, -1), W2)
    qq = _sc_propagate(z2, src2, dst2, cpt)
    out = _tc_post(qq[0], qq[1], z2, deg_a, deg_b, b2.reshape(1, -1),
                   Wh, bh.reshape(1, -1))
    return out


# speedup vs baseline: 1.0877x; 1.0877x over previous
import functools

import jax
import jax.numpy as jnp
from jax import lax
from jax.experimental import pallas as pl
from jax.experimental.pallas import tpu as pltpu
from jax.experimental.pallas import tpu_sc as plsc

_NC = 2
_NS = 16
_C2 = 112
_K = 3


def _pad_rows(n):
    g = _NS * 8
    return ((n + g - 1) // g) * g


def _pad_edges(a, fill, n_nodes):
    e = a.shape[0]
    nw = _NC * _NS
    assert e % nw == 0, e
    per = e // nw
    cpt = -(-per // (_C2 * 4)) * 4
    pad = cpt * _C2 - per
    a2 = a.reshape(nw, per)
    if pad:
        a2 = jnp.concatenate(
            [a2, jnp.full((nw, pad), fill, a.dtype)], axis=1)
    return a2.reshape(nw, cpt, _C2), cpt


def _sc_degree(dst2, cpt, n_nodes):
    n_pad = _pad_rows(n_nodes)
    rpt = n_pad // _NS

    mesh = plsc.VectorSubcoreMesh(core_axis_name="c", subcore_axis_name="s")

    @functools.partial(
        pl.kernel,
        out_type=jax.ShapeDtypeStruct((_NC * n_pad,), jnp.float32),
        mesh=mesh,
        scratch_types=[
            pltpu.VMEM((cpt, _C2), jnp.int32),
            pltpu.VMEM((_C2,), jnp.float32),
            pltpu.VMEM((rpt,), jnp.float32),
            pltpu.VMEM_SHARED((n_pad,), jnp.float32),
            pltpu.SemaphoreType.DMA,
        ],
    )
    def deg_kernel(dst_hbm, ones_h, zeros_h, out_hbm, didx_v, ones_v,
                   stage_v, acc_sh, sem):
        cid = lax.axis_index("c")
        sid = lax.axis_index("s")
        wid = cid * _NS + sid
        pltpu.sync_copy(zeros_h.at[pl.ds(sid * rpt, rpt)], stage_v)
        pltpu.sync_copy(stage_v, acc_sh.at[pl.ds(sid * rpt, rpt)])
        pltpu.sync_copy(ones_h, ones_v)
        pltpu.sync_copy(dst_hbm.at[wid], didx_v)
        plsc.subcore_barrier()

        def body(i, carry):
            descs = [
                pltpu.async_copy(ones_v, acc_sh.at[didx_v.at[i * 4 + k]],
                                 sem, add=True)
                for k in range(4)
            ]
            for desc in descs:
                desc.wait()
            return carry

        lax.fori_loop(0, cpt // 4, body, 0)
        plsc.subcore_barrier()
        pltpu.sync_copy(acc_sh.at[pl.ds(sid * rpt, rpt)], stage_v)
        pltpu.sync_copy(stage_v,
                        out_hbm.at[pl.ds(cid * n_pad + sid * rpt, rpt)])

    ones = jnp.ones((_C2,), jnp.float32)
    zeros = jnp.zeros((n_pad,), jnp.float32)
    return deg_kernel(dst2, ones, zeros)


def _sc_propagate(table, src2, dst2, cpt):
    n_nodes, d = table.shape
    n_pad = _pad_rows(n_nodes)
    rpt = n_pad // _NS
    full = n_nodes // rpt
    rem = n_nodes - full * rpt
    assert rem % 8 == 0

    mesh = plsc.VectorSubcoreMesh(core_axis_name="c", subcore_axis_name="s")

    @functools.partial(
        pl.kernel,
        out_type=jax.ShapeDtypeStruct((_NC, n_pad, d), jnp.float32),
        mesh=mesh,
        scratch_types=[
            pltpu.VMEM((cpt, _C2), jnp.int32),
            pltpu.VMEM((cpt, _C2), jnp.int32),
            pltpu.VMEM((_C2, d), jnp.float32),
            pltpu.VMEM_SHARED((n_pad, d), jnp.float32),
            pltpu.SemaphoreType.DMA,
        ],
    )
    def prop_kernel(table_hbm, src_hbm, dst_hbm, zeros_h, out_hbm,
                    sidx_v, didx_v, rows_v, acc_sh, semg):
        cid = lax.axis_index("c")
        sid = lax.axis_index("s")
        wid = cid * _NS + sid
        pltpu.sync_copy(zeros_h.at[pl.ds(sid * rpt, rpt)],
                        acc_sh.at[pl.ds(sid * rpt, rpt)])
        pltpu.sync_copy(src_hbm.at[wid], sidx_v)
        pltpu.sync_copy(dst_hbm.at[wid], didx_v)
        plsc.subcore_barrier()

        def body(j, carry):
            pltpu.async_copy(table_hbm.at[sidx_v.at[j]],
                             rows_v, semg).wait()
            pltpu.sync_copy(rows_v, acc_sh.at[didx_v.at[j]], add=True)
            return carry

        lax.fori_loop(0, cpt, body, 0)
        plsc.subcore_barrier()
        pltpu.sync_copy(acc_sh.at[pl.ds(sid * rpt, rpt)],
                        out_hbm.at[cid, pl.ds(sid * rpt, rpt)])

    zeros = jnp.zeros((n_pad, d), jnp.float32)
    return prop_kernel(table, src2, dst2, zeros)


_ROWS = 1000


def _tc_pre(x, deg0, deg1, w):
    n, d = x.shape
    dout = w.shape[1]
    assert n % _ROWS == 0

    def body(x_ref, d0_ref, d1_ref, w_ref, o_ref):
        dinv = lax.rsqrt(d0_ref[...] + d1_ref[...] + 1.0)
        o_ref[...] = jnp.dot(x_ref[...] * dinv, w_ref[...],
                             preferred_element_type=jnp.float32)

    return pl.pallas_call(
        body,
        grid=(n // _ROWS,),
        in_specs=[
            pl.BlockSpec((_ROWS, d), lambda i: (i, 0)),
            pl.BlockSpec((_ROWS, 1), lambda i: (i, 0)),
            pl.BlockSpec((_ROWS, 1), lambda i: (i, 0)),
            pl.BlockSpec((d, dout), lambda i: (0, 0)),
        ],
        out_specs=pl.BlockSpec((_ROWS, dout), lambda i: (i, 0)),
        out_shape=jax.ShapeDtypeStruct((n, dout), jnp.float32),
    )(x, deg0, deg1, w)


def _tc_mid(p0, p1, z, deg0, deg1, b, w):
    n, d = z.shape
    dout = w.shape[1]

    def body(p0_ref, p1_ref, z_ref, d0_ref, d1_ref, b_ref, w_ref, o_ref):
        dinv = lax.rsqrt(d0_ref[...] + d1_ref[...] + 1.0)
        pre = (p0_ref[...] + p1_ref[...] + z_ref[...]) * dinv + b_ref[...]
        h = jnp.maximum(pre, 0.0) * dinv
        o_ref[...] = jnp.dot(h, w_ref[...], preferred_element_type=jnp.float32)

    return pl.pallas_call(
        body,
        grid=(n // _ROWS,),
        in_specs=[
            pl.BlockSpec((_ROWS, d), lambda i: (i, 0)),
            pl.BlockSpec((_ROWS, d), lambda i: (i, 0)),
            pl.BlockSpec((_ROWS, d), lambda i: (i, 0)),
            pl.BlockSpec((_ROWS, 1), lambda i: (i, 0)),
            pl.BlockSpec((_ROWS, 1), lambda i: (i, 0)),
            pl.BlockSpec((1, d), lambda i: (0, 0)),
            pl.BlockSpec((d, dout), lambda i: (0, 0)),
        ],
        out_specs=pl.BlockSpec((_ROWS, dout), lambda i: (i, 0)),
        out_shape=jax.ShapeDtypeStruct((n, dout), jnp.float32),
    )(p0, p1, z, deg0, deg1, b, w)


def _tc_post(p0, p1, z, deg0, deg1, b, wh, bh):
    n, d = z.shape
    dout = wh.shape[1]

    def body(p0_ref, p1_ref, z_ref, d0_ref, d1_ref, b_ref, wh_ref, bh_ref,
             o_ref):
        dinv = lax.rsqrt(d0_ref[...] + d1_ref[...] + 1.0)
        pre = (p0_ref[...] + p1_ref[...] + z_ref[...]) * dinv + b_ref[...]
        h = jnp.maximum(pre, 0.0)
        o_ref[...] = jnp.dot(h, wh_ref[...],
                             preferred_element_type=jnp.float32) + bh_ref[...]

    return pl.pallas_call(
        body,
        grid=(n // _ROWS,),
        in_specs=[
            pl.BlockSpec((_ROWS, d), lambda i: (i, 0)),
            pl.BlockSpec((_ROWS, d), lambda i: (i, 0)),
            pl.BlockSpec((_ROWS, d), lambda i: (i, 0)),
            pl.BlockSpec((_ROWS, 1), lambda i: (i, 0)),
            pl.BlockSpec((_ROWS, 1), lambda i: (i, 0)),
            pl.BlockSpec((1, d), lambda i: (0, 0)),
            pl.BlockSpec((d, dout), lambda i: (0, 0)),
            pl.BlockSpec((1, dout), lambda i: (0, 0)),
        ],
        out_specs=pl.BlockSpec((_ROWS, dout), lambda i: (i, 0)),
        out_shape=jax.ShapeDtypeStruct((n, dout), jnp.float32),
    )(p0, p1, z, deg0, deg1, b, wh, bh)


def kernel(x, edge_index, W1, b1, W2, b2, Wh, bh):
    n, d = x.shape
    src = edge_index[0].astype(jnp.int32)
    dst = edge_index[1].astype(jnp.int32)
    src2, cpt = _pad_edges(src, 0, n)
    dst2, _ = _pad_edges(dst, n, n)

    n_pad = _pad_rows(n)
    deg2 = _sc_degree(dst2, cpt, n).reshape(_NC, n_pad, 1)
    deg_a, deg_b = deg2[0], deg2[1]

    z1 = _tc_pre(x, deg_a, deg_b, W1)
    pp = _sc_propagate(z1, src2, dst2, cpt)
    z2 = _tc_mid(pp[0], pp[1], z1, deg_a, deg_b, b1.reshape(1, -1), W2)
    qq = _sc_propagate(z2, src2, dst2, cpt)
    out = _tc_post(qq[0], qq[1], z2, deg_a, deg_b, b2.reshape(1, -1),
                   Wh, bh.reshape(1, -1))
    return out


# speedup vs baseline: 2.3374x; 2.1488x over previous
import functools

import jax
import jax.numpy as jnp
from jax import lax
from jax.experimental import pallas as pl
from jax.experimental.pallas import tpu as pltpu
from jax.experimental.pallas import tpu_sc as plsc

_NC = 2
_NS = 16
_C2 = 112
_K = 3
_CHUNK = 80


def _pad_rows(n):
    g = _NS * 8
    return ((n + g - 1) // g) * g


def _pad_edges(a, fill, n_nodes):
    e = a.shape[0]
    nw = _NC * _NS
    assert e % nw == 0, e
    per = e // nw
    cpt = -(-per // (_C2 * 4)) * 4
    pad = cpt * _C2 - per
    a2 = a.reshape(nw, per)
    if pad:
        a2 = jnp.concatenate(
            [a2, jnp.full((nw, pad), fill, a.dtype)], axis=1)
    return a2.reshape(nw, cpt, _C2), cpt


def _sc_degree(dst2, cpt, n_nodes):
    n_pad = _pad_rows(n_nodes)
    rpt = n_pad // _NS

    mesh = plsc.VectorSubcoreMesh(core_axis_name="c", subcore_axis_name="s")

    @functools.partial(
        pl.kernel,
        out_type=jax.ShapeDtypeStruct((_NC * n_pad,), jnp.float32),
        mesh=mesh,
        scratch_types=[
            pltpu.VMEM((cpt, _C2), jnp.int32),
            pltpu.VMEM((_C2,), jnp.float32),
            pltpu.VMEM((rpt,), jnp.float32),
            pltpu.VMEM_SHARED((n_pad,), jnp.float32),
            pltpu.SemaphoreType.DMA,
        ],
    )
    def deg_kernel(dst_hbm, ones_h, zeros_h, out_hbm, didx_v, ones_v,
                   stage_v, acc_sh, sem):
        cid = lax.axis_index("c")
        sid = lax.axis_index("s")
        wid = cid * _NS + sid
        pltpu.sync_copy(zeros_h.at[pl.ds(sid * rpt, rpt)], stage_v)
        pltpu.sync_copy(stage_v, acc_sh.at[pl.ds(sid * rpt, rpt)])
        pltpu.sync_copy(ones_h, ones_v)
        pltpu.sync_copy(dst_hbm.at[wid], didx_v)
        plsc.subcore_barrier()

        def body(i, carry):
            descs = [
                pltpu.async_copy(ones_v, acc_sh.at[didx_v.at[i * 4 + k]],
                                 sem, add=True)
                for k in range(4)
            ]
            for desc in descs:
                desc.wait()
            return carry

        lax.fori_loop(0, cpt // 4, body, 0)
        plsc.subcore_barrier()
        pltpu.sync_copy(acc_sh.at[pl.ds(sid * rpt, rpt)], stage_v)
        pltpu.sync_copy(stage_v,
                        out_hbm.at[pl.ds(cid * n_pad + sid * rpt, rpt)])

    ones = jnp.ones((_C2,), jnp.float32)
    zeros = jnp.zeros((n_pad,), jnp.float32)
    return deg_kernel(dst2, ones, zeros)


def _sc_propagate(table, src_i32, dst_i32):
    n_nodes, d = table.shape
    e = src_i32.shape[0]
    nw = _NC * _NS
    assert e % (nw * _CHUNK) == 0, e
    per_tile = e // nw
    cpt = per_tile // _CHUNK
    n_pad = _pad_rows(n_nodes)
    rpt = n_pad // _NS
    pairs = cpt // 2
    tail = cpt - 2 * pairs

    mesh = plsc.VectorSubcoreMesh(core_axis_name="c", subcore_axis_name="s")

    @functools.partial(
        pl.kernel,
        out_type=jax.ShapeDtypeStruct((_NC, n_pad, d), jnp.float32),
        mesh=mesh,
        scratch_types=[
            [pltpu.VMEM((_CHUNK,), jnp.int32)] * 2,
            [pltpu.VMEM((_CHUNK,), jnp.int32)] * 2,
            [pltpu.VMEM((_CHUNK, d), jnp.float32)] * 2,
            pltpu.VMEM_SHARED((n_pad, d), jnp.float32),
            [pltpu.SemaphoreType.DMA] * 2,
        ],
    )
    def prop_kernel(table_hbm, src_hbm, dst_hbm, zeros_h, out_hbm,
                    sidx_v, didx_v, rows_v, acc_sh, semg):
        cid = lax.axis_index("c")
        sid = lax.axis_index("s")
        wid = cid * _NS + sid
        pltpu.sync_copy(zeros_h.at[pl.ds(sid * rpt, rpt)],
                        acc_sh.at[pl.ds(sid * rpt, rpt)])
        plsc.subcore_barrier()
        base0 = wid * per_tile

        def load_and_gather(j, k):
            base = base0 + j * _CHUNK
            pltpu.sync_copy(src_hbm.at[pl.ds(base, _CHUNK)], sidx_v[k])
            pltpu.sync_copy(dst_hbm.at[pl.ds(base, _CHUNK)], didx_v[k])
            return pltpu.async_copy(table_hbm.at[sidx_v[k]],
                                    rows_v[k], semg[k])

        def body(i, carry):
            g0 = load_and_gather(2 * i, 0)
            g1 = load_and_gather(2 * i + 1, 1)
            g0.wait()
            pltpu.sync_copy(rows_v[0], acc_sh.at[didx_v[0]], add=True)
            g1.wait()
            pltpu.sync_copy(rows_v[1], acc_sh.at[didx_v[1]], add=True)
            return carry

        lax.fori_loop(0, pairs, body, 0)
        if tail:
            g0 = load_and_gather(cpt - 1, 0)
            g0.wait()
            pltpu.sync_copy(rows_v[0], acc_sh.at[didx_v[0]], add=True)
        plsc.subcore_barrier()
        pltpu.sync_copy(acc_sh.at[pl.ds(sid * rpt, rpt)],
                        out_hbm.at[cid, pl.ds(sid * rpt, rpt)])

    zeros = jnp.zeros((n_pad, d), jnp.float32)
    return prop_kernel(table, src_i32, dst_i32, zeros)


_ROWS = 1000


def _tc_pre(x, deg0, deg1, w):
    n, d = x.shape
    dout = w.shape[1]
    assert n % _ROWS == 0

    def body(x_ref, d0_ref, d1_ref, w_ref, o_ref):
        dinv = lax.rsqrt(d0_ref[...] + d1_ref[...] + 1.0)
        o_ref[...] = jnp.dot(x_ref[...] * dinv, w_ref[...],
                             preferred_element_type=jnp.float32)

    return pl.pallas_call(
        body,
        grid=(n // _ROWS,),
        in_specs=[
            pl.BlockSpec((_ROWS, d), lambda i: (i, 0)),
            pl.BlockSpec((_ROWS, 1), lambda i: (i, 0)),
            pl.BlockSpec((_ROWS, 1), lambda i: (i, 0)),
            pl.BlockSpec((d, dout), lambda i: (0, 0)),
        ],
        out_specs=pl.BlockSpec((_ROWS, dout), lambda i: (i, 0)),
        out_shape=jax.ShapeDtypeStruct((n, dout), jnp.float32),
    )(x, deg0, deg1, w)


def _tc_mid(p0, p1, z, deg0, deg1, b, w):
    n, d = z.shape
    dout = w.shape[1]

    def body(p0_ref, p1_ref, z_ref, d0_ref, d1_ref, b_ref, w_ref, o_ref):
        dinv = lax.rsqrt(d0_ref[...] + d1_ref[...] + 1.0)
        pre = (p0_ref[...] + p1_ref[...] + z_ref[...]) * dinv + b_ref[...]
        h = jnp.maximum(pre, 0.0) * dinv
        o_ref[...] = jnp.dot(h, w_ref[...], preferred_element_type=jnp.float32)

    return pl.pallas_call(
        body,
        grid=(n // _ROWS,),
        in_specs=[
            pl.BlockSpec((_ROWS, d), lambda i: (i, 0)),
            pl.BlockSpec((_ROWS, d), lambda i: (i, 0)),
            pl.BlockSpec((_ROWS, d), lambda i: (i, 0)),
            pl.BlockSpec((_ROWS, 1), lambda i: (i, 0)),
            pl.BlockSpec((_ROWS, 1), lambda i: (i, 0)),
            pl.BlockSpec((1, d), lambda i: (0, 0)),
            pl.BlockSpec((d, dout), lambda i: (0, 0)),
        ],
        out_specs=pl.BlockSpec((_ROWS, dout), lambda i: (i, 0)),
        out_shape=jax.ShapeDtypeStruct((n, dout), jnp.float32),
    )(p0, p1, z, deg0, deg1, b, w)


def _tc_post(p0, p1, z, deg0, deg1, b, wh, bh):
    n, d = z.shape
    dout = wh.shape[1]

    def body(p0_ref, p1_ref, z_ref, d0_ref, d1_ref, b_ref, wh_ref, bh_ref,
             o_ref):
        dinv = lax.rsqrt(d0_ref[...] + d1_ref[...] + 1.0)
        pre = (p0_ref[...] + p1_ref[...] + z_ref[...]) * dinv + b_ref[...]
        h = jnp.maximum(pre, 0.0)
        o_ref[...] = jnp.dot(h, wh_ref[...],
                             preferred_element_type=jnp.float32) + bh_ref[...]

    return pl.pallas_call(
        body,
        grid=(n // _ROWS,),
        in_specs=[
            pl.BlockSpec((_ROWS, d), lambda i: (i, 0)),
            pl.BlockSpec((_ROWS, d), lambda i: (i, 0)),
            pl.BlockSpec((_ROWS, d), lambda i: (i, 0)),
            pl.BlockSpec((_ROWS, 1), lambda i: (i, 0)),
            pl.BlockSpec((_ROWS, 1), lambda i: (i, 0)),
            pl.BlockSpec((1, d), lambda i: (0, 0)),
            pl.BlockSpec((d, dout), lambda i: (0, 0)),
            pl.BlockSpec((1, dout), lambda i: (0, 0)),
        ],
        out_specs=pl.BlockSpec((_ROWS, dout), lambda i: (i, 0)),
        out_shape=jax.ShapeDtypeStruct((n, dout), jnp.float32),
    )(p0, p1, z, deg0, deg1, b, wh, bh)


def kernel(x, edge_index, W1, b1, W2, b2, Wh, bh):
    n, d = x.shape
    src = edge_index[0].astype(jnp.int32)
    dst = edge_index[1].astype(jnp.int32)
    src2, cpt = _pad_edges(src, 0, n)
    dst2, _ = _pad_edges(dst, n, n)

    n_pad = _pad_rows(n)
    deg2 = _sc_degree(dst2, cpt, n).reshape(_NC, n_pad, 1)
    deg_a, deg_b = deg2[0], deg2[1]

    z1 = _tc_pre(x, deg_a, deg_b, W1)
    pp = _sc_propagate(z1, src, dst)
    z2 = _tc_mid(pp[0], pp[1], z1, deg_a, deg_b, b1.reshape(1, -1), W2)
    qq = _sc_propagate(z2, src, dst)
    out = _tc_post(qq[0], qq[1], z2, deg_a, deg_b, b2.reshape(1, -1),
                   Wh, bh.reshape(1, -1))
    return out


# speedup vs baseline: 2.6120x; 1.1175x over previous
import functools

import jax
import jax.numpy as jnp
from jax import lax
from jax.experimental import pallas as pl
from jax.experimental.pallas import tpu as pltpu
from jax.experimental.pallas import tpu_sc as plsc

_NC = 2
_NS = 16
_C2 = 112
_K = 3
_CHUNK = 80


def _pad_rows(n):
    g = _NS * 8
    return ((n + g - 1) // g) * g


def _pad_edges(a, fill, n_nodes):
    e = a.shape[0]
    nw = _NC * _NS
    assert e % nw == 0, e
    per = e // nw
    cpt = -(-per // (_C2 * 4)) * 4
    pad = cpt * _C2 - per
    a2 = a.reshape(nw, per)
    if pad:
        a2 = jnp.concatenate(
            [a2, jnp.full((nw, pad), fill, a.dtype)], axis=1)
    return a2.reshape(nw, cpt, _C2), cpt


def _sc_degree(dst2, cpt, n_nodes):
    n_pad = _pad_rows(n_nodes)
    rpt = n_pad // _NS

    mesh = plsc.VectorSubcoreMesh(core_axis_name="c", subcore_axis_name="s")

    @functools.partial(
        pl.kernel,
        out_type=jax.ShapeDtypeStruct((_NC * n_pad,), jnp.float32),
        mesh=mesh,
        scratch_types=[
            pltpu.VMEM((cpt, _C2), jnp.int32),
            pltpu.VMEM((_C2,), jnp.float32),
            pltpu.VMEM((rpt,), jnp.float32),
            pltpu.VMEM_SHARED((n_pad,), jnp.float32),
            pltpu.SemaphoreType.DMA,
        ],
    )
    def deg_kernel(dst_hbm, ones_h, zeros_h, out_hbm, didx_v, ones_v,
                   stage_v, acc_sh, sem):
        cid = lax.axis_index("c")
        sid = lax.axis_index("s")
        wid = cid * _NS + sid
        pltpu.sync_copy(zeros_h.at[pl.ds(sid * rpt, rpt)], stage_v)
        pltpu.sync_copy(stage_v, acc_sh.at[pl.ds(sid * rpt, rpt)])
        pltpu.sync_copy(ones_h, ones_v)
        pltpu.sync_copy(dst_hbm.at[wid], didx_v)
        plsc.subcore_barrier()

        def body(i, carry):
            descs = [
                pltpu.async_copy(ones_v, acc_sh.at[didx_v.at[i * 4 + k]],
                                 sem, add=True)
                for k in range(4)
            ]
            for desc in descs:
                desc.wait()
            return carry

        lax.fori_loop(0, cpt // 4, body, 0)
        plsc.subcore_barrier()
        pltpu.sync_copy(acc_sh.at[pl.ds(sid * rpt, rpt)], stage_v)
        pltpu.sync_copy(stage_v,
                        out_hbm.at[pl.ds(cid * n_pad + sid * rpt, rpt)])

    ones = jnp.ones((_C2,), jnp.float32)
    zeros = jnp.zeros((n_pad,), jnp.float32)
    return deg_kernel(dst2, ones, zeros)


def _sc_propagate(table, src_i32, dst_i32):
    n_nodes, d = table.shape
    e = src_i32.shape[0]
    nw = _NC * _NS
    assert e % (nw * _CHUNK) == 0, e
    per_tile = e // nw
    cpt = per_tile // _CHUNK
    n_pad = _pad_rows(n_nodes)
    rpt = n_pad // _NS
    nb = 3
    groups = cpt // nb
    tail = cpt - nb * groups

    mesh = plsc.VectorSubcoreMesh(core_axis_name="c", subcore_axis_name="s")

    @functools.partial(
        pl.kernel,
        out_type=jax.ShapeDtypeStruct((_NC, n_pad, d), jnp.float32),
        mesh=mesh,
        scratch_types=[
            [pltpu.VMEM((_CHUNK,), jnp.int32)] * 3,
            [pltpu.VMEM((_CHUNK,), jnp.int32)] * 3,
            [pltpu.VMEM((_CHUNK, d), jnp.float32)] * 3,
            pltpu.VMEM_SHARED((n_pad, d), jnp.float32),
            [pltpu.SemaphoreType.DMA] * 3,
        ],
    )
    def prop_kernel(table_hbm, src_hbm, dst_hbm, zeros_h, out_hbm,
                    sidx_v, didx_v, rows_v, acc_sh, semg):
        cid = lax.axis_index("c")
        sid = lax.axis_index("s")
        wid = cid * _NS + sid
        pltpu.sync_copy(zeros_h.at[pl.ds(sid * rpt, rpt)],
                        acc_sh.at[pl.ds(sid * rpt, rpt)])
        plsc.subcore_barrier()
        base0 = wid * per_tile

        def load_and_gather(j, k):
            base = base0 + j * _CHUNK
            pltpu.sync_copy(src_hbm.at[pl.ds(base, _CHUNK)], sidx_v[k])
            pltpu.sync_copy(dst_hbm.at[pl.ds(base, _CHUNK)], didx_v[k])
            return pltpu.async_copy(table_hbm.at[sidx_v[k]],
                                    rows_v[k], semg[k])

        def body(i, carry):
            gs = [load_and_gather(nb * i + k, k) for k in range(nb)]
            for k in range(nb):
                gs[k].wait()
                pltpu.sync_copy(rows_v[k], acc_sh.at[didx_v[k]], add=True)
            return carry

        lax.fori_loop(0, groups, body, 0)
        if tail:
            gs = [load_and_gather(nb * groups + k, k) for k in range(tail)]
            for k in range(tail):
                gs[k].wait()
                pltpu.sync_copy(rows_v[k], acc_sh.at[didx_v[k]], add=True)
        plsc.subcore_barrier()
        pltpu.sync_copy(acc_sh.at[pl.ds(sid * rpt, rpt)],
                        out_hbm.at[cid, pl.ds(sid * rpt, rpt)])

    zeros = jnp.zeros((n_pad, d), jnp.float32)
    return prop_kernel(table, src_i32, dst_i32, zeros)


_ROWS = 1000


def _tc_pre(x, deg0, deg1, w):
    n, d = x.shape
    dout = w.shape[1]
    assert n % _ROWS == 0

    def body(x_ref, d0_ref, d1_ref, w_ref, o_ref):
        dinv = lax.rsqrt(d0_ref[...] + d1_ref[...] + 1.0)
        o_ref[...] = jnp.dot(x_ref[...] * dinv, w_ref[...],
                             preferred_element_type=jnp.float32)

    return pl.pallas_call(
        body,
        grid=(n // _ROWS,),
        in_specs=[
            pl.BlockSpec((_ROWS, d), lambda i: (i, 0)),
            pl.BlockSpec((_ROWS, 1), lambda i: (i, 0)),
            pl.BlockSpec((_ROWS, 1), lambda i: (i, 0)),
            pl.BlockSpec((d, dout), lambda i: (0, 0)),
        ],
        out_specs=pl.BlockSpec((_ROWS, dout), lambda i: (i, 0)),
        out_shape=jax.ShapeDtypeStruct((n, dout), jnp.float32),
    )(x, deg0, deg1, w)


def _tc_mid(p0, p1, z, deg0, deg1, b, w):
    n, d = z.shape
    dout = w.shape[1]

    def body(p0_ref, p1_ref, z_ref, d0_ref, d1_ref, b_ref, w_ref, o_ref):
        dinv = lax.rsqrt(d0_ref[...] + d1_ref[...] + 1.0)
        pre = (p0_ref[...] + p1_ref[...] + z_ref[...]) * dinv + b_ref[...]
        h = jnp.maximum(pre, 0.0) * dinv
        o_ref[...] = jnp.dot(h, w_ref[...], preferred_element_type=jnp.float32)

    return pl.pallas_call(
        body,
        grid=(n // _ROWS,),
        in_specs=[
            pl.BlockSpec((_ROWS, d), lambda i: (i, 0)),
            pl.BlockSpec((_ROWS, d), lambda i: (i, 0)),
            pl.BlockSpec((_ROWS, d), lambda i: (i, 0)),
            pl.BlockSpec((_ROWS, 1), lambda i: (i, 0)),
            pl.BlockSpec((_ROWS, 1), lambda i: (i, 0)),
            pl.BlockSpec((1, d), lambda i: (0, 0)),
            pl.BlockSpec((d, dout), lambda i: (0, 0)),
        ],
        out_specs=pl.BlockSpec((_ROWS, dout), lambda i: (i, 0)),
        out_shape=jax.ShapeDtypeStruct((n, dout), jnp.float32),
    )(p0, p1, z, deg0, deg1, b, w)


def _tc_post(p0, p1, z, deg0, deg1, b, wh, bh):
    n, d = z.shape
    dout = wh.shape[1]

    def body(p0_ref, p1_ref, z_ref, d0_ref, d1_ref, b_ref, wh_ref, bh_ref,
             o_ref):
        dinv = lax.rsqrt(d0_ref[...] + d1_ref[...] + 1.0)
        pre = (p0_ref[...] + p1_ref[...] + z_ref[...]) * dinv + b_ref[...]
        h = jnp.maximum(pre, 0.0)
        o_ref[...] = jnp.dot(h, wh_ref[...],
                             preferred_element_type=jnp.float32) + bh_ref[...]

    return pl.pallas_call(
        body,
        grid=(n // _ROWS,),
        in_specs=[
            pl.BlockSpec((_ROWS, d), lambda i: (i, 0)),
            pl.BlockSpec((_ROWS, d), lambda i: (i, 0)),
            pl.BlockSpec((_ROWS, d), lambda i: (i, 0)),
            pl.BlockSpec((_ROWS, 1), lambda i: (i, 0)),
            pl.BlockSpec((_ROWS, 1), lambda i: (i, 0)),
            pl.BlockSpec((1, d), lambda i: (0, 0)),
            pl.BlockSpec((d, dout), lambda i: (0, 0)),
            pl.BlockSpec((1, dout), lambda i: (0, 0)),
        ],
        out_specs=pl.BlockSpec((_ROWS, dout), lambda i: (i, 0)),
        out_shape=jax.ShapeDtypeStruct((n, dout), jnp.float32),
    )(p0, p1, z, deg0, deg1, b, wh, bh)


def kernel(x, edge_index, W1, b1, W2, b2, Wh, bh):
    n, d = x.shape
    src = edge_index[0].astype(jnp.int32)
    dst = edge_index[1].astype(jnp.int32)
    src2, cpt = _pad_edges(src, 0, n)
    dst2, _ = _pad_edges(dst, n, n)

    n_pad = _pad_rows(n)
    deg2 = _sc_degree(dst2, cpt, n).reshape(_NC, n_pad, 1)
    deg_a, deg_b = deg2[0], deg2[1]

    z1 = _tc_pre(x, deg_a, deg_b, W1)
    pp = _sc_propagate(z1, src, dst)
    z2 = _tc_mid(pp[0], pp[1], z1, deg_a, deg_b, b1.reshape(1, -1), W2)
    qq = _sc_propagate(z2, src, dst)
    out = _tc_post(qq[0], qq[1], z2, deg_a, deg_b, b2.reshape(1, -1),
                   Wh, bh.reshape(1, -1))
    return out


# speedup vs baseline: 2.6881x; 1.0291x over previous
import functools

import jax
import jax.numpy as jnp
from jax import lax
from jax.experimental import pallas as pl
from jax.experimental.pallas import tpu as pltpu
from jax.experimental.pallas import tpu_sc as plsc

_NC = 2
_NS = 16
_C2 = 112
_K = 3
_CHUNK = 80


def _pad_rows(n):
    g = _NS * 8
    return ((n + g - 1) // g) * g


def _pad_edges(a, fill, n_nodes):
    e = a.shape[0]
    nw = _NC * _NS
    assert e % nw == 0, e
    per = e // nw
    cpt = -(-per // (_C2 * 4)) * 4
    pad = cpt * _C2 - per
    a2 = a.reshape(nw, per)
    if pad:
        a2 = jnp.concatenate(
            [a2, jnp.full((nw, pad), fill, a.dtype)], axis=1)
    return a2.reshape(nw, cpt, _C2), cpt


def _sc_degree(dst2, cpt, n_nodes):
    n_pad = _pad_rows(n_nodes)
    rpt = n_pad // _NS

    mesh = plsc.VectorSubcoreMesh(core_axis_name="c", subcore_axis_name="s")

    @functools.partial(
        pl.kernel,
        out_type=jax.ShapeDtypeStruct((_NC * n_pad,), jnp.float32),
        mesh=mesh,
        scratch_types=[
            pltpu.VMEM((cpt, _C2), jnp.int32),
            pltpu.VMEM((_C2,), jnp.float32),
            pltpu.VMEM((rpt,), jnp.float32),
            pltpu.VMEM_SHARED((n_pad,), jnp.float32),
            pltpu.SemaphoreType.DMA,
        ],
    )
    def deg_kernel(dst_hbm, ones_h, zeros_h, out_hbm, didx_v, ones_v,
                   stage_v, acc_sh, sem):
        cid = lax.axis_index("c")
        sid = lax.axis_index("s")
        wid = cid * _NS + sid
        pltpu.sync_copy(zeros_h.at[pl.ds(sid * rpt, rpt)], stage_v)
        pltpu.sync_copy(stage_v, acc_sh.at[pl.ds(sid * rpt, rpt)])
        pltpu.sync_copy(ones_h, ones_v)
        pltpu.sync_copy(dst_hbm.at[wid], didx_v)
        plsc.subcore_barrier()

        def body(i, carry):
            descs = [
                pltpu.async_copy(ones_v, acc_sh.at[didx_v.at[i * 4 + k]],
                                 sem, add=True)
                for k in range(4)
            ]
            for desc in descs:
                desc.wait()
            return carry

        lax.fori_loop(0, cpt // 4, body, 0)
        plsc.subcore_barrier()
        pltpu.sync_copy(acc_sh.at[pl.ds(sid * rpt, rpt)], stage_v)
        pltpu.sync_copy(stage_v,
                        out_hbm.at[pl.ds(cid * n_pad + sid * rpt, rpt)])

    ones = jnp.ones((_C2,), jnp.float32)
    zeros = jnp.zeros((n_pad,), jnp.float32)
    return deg_kernel(dst2, ones, zeros)


def _sc_propagate(table, src_i32, dst_i32):
    n_nodes, d = table.shape
    e = src_i32.shape[0]
    nw = _NC * _NS
    assert e % (nw * _CHUNK) == 0, e
    per_tile = e // nw
    cpt = per_tile // _CHUNK
    n_pad = _pad_rows(n_nodes)
    rpt = n_pad // _NS
    nb = 4
    groups = cpt // nb
    tail = cpt - nb * groups

    mesh = plsc.VectorSubcoreMesh(core_axis_name="c", subcore_axis_name="s")

    @functools.partial(
        pl.kernel,
        out_type=jax.ShapeDtypeStruct((_NC, n_pad, d), jnp.float32),
        mesh=mesh,
        scratch_types=[
            [pltpu.VMEM((_CHUNK,), jnp.int32)] * 4,
            [pltpu.VMEM((_CHUNK,), jnp.int32)] * 4,
            [pltpu.VMEM((_CHUNK, d), jnp.float32)] * 4,
            pltpu.VMEM_SHARED((n_pad, d), jnp.float32),
            [pltpu.SemaphoreType.DMA] * 4,
        ],
    )
    def prop_kernel(table_hbm, src_hbm, dst_hbm, zeros_h, out_hbm,
                    sidx_v, didx_v, rows_v, acc_sh, semg):
        cid = lax.axis_index("c")
        sid = lax.axis_index("s")
        wid = cid * _NS + sid
        pltpu.sync_copy(zeros_h.at[pl.ds(sid * rpt, rpt)],
                        acc_sh.at[pl.ds(sid * rpt, rpt)])
        plsc.subcore_barrier()
        base0 = wid * per_tile

        def load_and_gather(j, k):
            base = base0 + j * _CHUNK
            pltpu.sync_copy(src_hbm.at[pl.ds(base, _CHUNK)], sidx_v[k])
            pltpu.sync_copy(dst_hbm.at[pl.ds(base, _CHUNK)], didx_v[k])
            return pltpu.async_copy(table_hbm.at[sidx_v[k]],
                                    rows_v[k], semg[k])

        def body(i, carry):
            gs = [load_and_gather(nb * i + k, k) for k in range(nb)]
            for k in range(nb):
                gs[k].wait()
                pltpu.sync_copy(rows_v[k], acc_sh.at[didx_v[k]], add=True)
            return carry

        lax.fori_loop(0, groups, body, 0)
        if tail:
            gs = [load_and_gather(nb * groups + k, k) for k in range(tail)]
            for k in range(tail):
                gs[k].wait()
                pltpu.sync_copy(rows_v[k], acc_sh.at[didx_v[k]], add=True)
        plsc.subcore_barrier()
        pltpu.sync_copy(acc_sh.at[pl.ds(sid * rpt, rpt)],
                        out_hbm.at[cid, pl.ds(sid * rpt, rpt)])

    zeros = jnp.zeros((n_pad, d), jnp.float32)
    return prop_kernel(table, src_i32, dst_i32, zeros)


_ROWS = 1000


def _tc_pre(x, deg0, deg1, w):
    n, d = x.shape
    dout = w.shape[1]
    assert n % _ROWS == 0

    def body(x_ref, d0_ref, d1_ref, w_ref, o_ref):
        dinv = lax.rsqrt(d0_ref[...] + d1_ref[...] + 1.0)
        o_ref[...] = jnp.dot(x_ref[...] * dinv, w_ref[...],
                             preferred_element_type=jnp.float32)

    return pl.pallas_call(
        body,
        grid=(n // _ROWS,),
        in_specs=[
            pl.BlockSpec((_ROWS, d), lambda i: (i, 0)),
            pl.BlockSpec((_ROWS, 1), lambda i: (i, 0)),
            pl.BlockSpec((_ROWS, 1), lambda i: (i, 0)),
            pl.BlockSpec((d, dout), lambda i: (0, 0)),
        ],
        out_specs=pl.BlockSpec((_ROWS, dout), lambda i: (i, 0)),
        out_shape=jax.ShapeDtypeStruct((n, dout), jnp.float32),
    )(x, deg0, deg1, w)


def _tc_mid(p0, p1, z, deg0, deg1, b, w):
    n, d = z.shape
    dout = w.shape[1]

    def body(p0_ref, p1_ref, z_ref, d0_ref, d1_ref, b_ref, w_ref, o_ref):
        dinv = lax.rsqrt(d0_ref[...] + d1_ref[...] + 1.0)
        pre = (p0_ref[...] + p1_ref[...] + z_ref[...]) * dinv + b_ref[...]
        h = jnp.maximum(pre, 0.0) * dinv
        o_ref[...] = jnp.dot(h, w_ref[...], preferred_element_type=jnp.float32)

    return pl.pallas_call(
        body,
        grid=(n // _ROWS,),
        in_specs=[
            pl.BlockSpec((_ROWS, d), lambda i: (i, 0)),
            pl.BlockSpec((_ROWS, d), lambda i: (i, 0)),
            pl.BlockSpec((_ROWS, d), lambda i: (i, 0)),
            pl.BlockSpec((_ROWS, 1), lambda i: (i, 0)),
            pl.BlockSpec((_ROWS, 1), lambda i: (i, 0)),
            pl.BlockSpec((1, d), lambda i: (0, 0)),
            pl.BlockSpec((d, dout), lambda i: (0, 0)),
        ],
        out_specs=pl.BlockSpec((_ROWS, dout), lambda i: (i, 0)),
        out_shape=jax.ShapeDtypeStruct((n, dout), jnp.float32),
    )(p0, p1, z, deg0, deg1, b, w)


def _tc_post(p0, p1, z, deg0, deg1, b, wh, bh):
    n, d = z.shape
    dout = wh.shape[1]

    def body(p0_ref, p1_ref, z_ref, d0_ref, d1_ref, b_ref, wh_ref, bh_ref,
             o_ref):
        dinv = lax.rsqrt(d0_ref[...] + d1_ref[...] + 1.0)
        pre = (p0_ref[...] + p1_ref[...] + z_ref[...]) * dinv + b_ref[...]
        h = jnp.maximum(pre, 0.0)
        o_ref[...] = jnp.dot(h, wh_ref[...],
                             preferred_element_type=jnp.float32) + bh_ref[...]

    return pl.pallas_call(
        body,
        grid=(n // _ROWS,),
        in_specs=[
            pl.BlockSpec((_ROWS, d), lambda i: (i, 0)),
            pl.BlockSpec((_ROWS, d), lambda i: (i, 0)),
            pl.BlockSpec((_ROWS, d), lambda i: (i, 0)),
            pl.BlockSpec((_ROWS, 1), lambda i: (i, 0)),
            pl.BlockSpec((_ROWS, 1), lambda i: (i, 0)),
            pl.BlockSpec((1, d), lambda i: (0, 0)),
            pl.BlockSpec((d, dout), lambda i: (0, 0)),
            pl.BlockSpec((1, dout), lambda i: (0, 0)),
        ],
        out_specs=pl.BlockSpec((_ROWS, dout), lambda i: (i, 0)),
        out_shape=jax.ShapeDtypeStruct((n, dout), jnp.float32),
    )(p0, p1, z, deg0, deg1, b, wh, bh)


def kernel(x, edge_index, W1, b1, W2, b2, Wh, bh):
    n, d = x.shape
    src = edge_index[0].astype(jnp.int32)
    dst = edge_index[1].astype(jnp.int32)
    src2, cpt = _pad_edges(src, 0, n)
    dst2, _ = _pad_edges(dst, n, n)

    n_pad = _pad_rows(n)
    deg2 = _sc_degree(dst2, cpt, n).reshape(_NC, n_pad, 1)
    deg_a, deg_b = deg2[0], deg2[1]

    z1 = _tc_pre(x, deg_a, deg_b, W1)
    pp = _sc_propagate(z1, src, dst)
    z2 = _tc_mid(pp[0], pp[1], z1, deg_a, deg_b, b1.reshape(1, -1), W2)
    qq = _sc_propagate(z2, src, dst)
    out = _tc_post(qq[0], qq[1], z2, deg_a, deg_b, b2.reshape(1, -1),
                   Wh, bh.reshape(1, -1))
    return out


# speedup vs baseline: 2.6886x; 1.0002x over previous
import functools

import jax
import jax.numpy as jnp
from jax import lax
from jax.experimental import pallas as pl
from jax.experimental.pallas import tpu as pltpu
from jax.experimental.pallas import tpu_sc as plsc

_NC = 2
_NS = 16
_C2 = 112
_CHUNK = 80


def _pad_rows(n):
    g = _NS * 8
    return ((n + g - 1) // g) * g


def _pad_edges(a, fill):
    e = a.shape[0]
    nw = _NC * _NS
    assert e % nw == 0, e
    per = e // nw
    cpt = -(-per // (_C2 * 4)) * 4
    pad = cpt * _C2 - per
    a2 = a.reshape(nw, per)
    if pad:
        a2 = jnp.concatenate(
            [a2, jnp.full((nw, pad), fill, a.dtype)], axis=1)
    return a2.reshape(nw, cpt, _C2), cpt


def _sc_degree(dst2, cpt, n_nodes):
    n_pad = _pad_rows(n_nodes)
    rpt = n_pad // _NS

    mesh = plsc.VectorSubcoreMesh(core_axis_name="c", subcore_axis_name="s")

    @functools.partial(
        pl.kernel,
        out_type=jax.ShapeDtypeStruct((_NC * n_pad,), jnp.float32),
        mesh=mesh,
        scratch_types=[
            pltpu.VMEM((cpt, _C2), jnp.int32),
            pltpu.VMEM((_C2,), jnp.float32),
            pltpu.VMEM((rpt,), jnp.float32),
            pltpu.VMEM_SHARED((n_pad,), jnp.float32),
            pltpu.SemaphoreType.DMA,
        ],
    )
    def deg_kernel(dst_hbm, ones_h, zeros_h, out_hbm, didx_v, ones_v,
                   stage_v, acc_sh, sem):
        cid = lax.axis_index("c")
        sid = lax.axis_index("s")
        wid = cid * _NS + sid
        pltpu.sync_copy(zeros_h.at[pl.ds(sid * rpt, rpt)], stage_v)
        pltpu.sync_copy(stage_v, acc_sh.at[pl.ds(sid * rpt, rpt)])
        pltpu.sync_copy(ones_h, ones_v)
        pltpu.sync_copy(dst_hbm.at[wid], didx_v)
        plsc.subcore_barrier()

        def body(i, carry):
            descs = [
                pltpu.async_copy(ones_v, acc_sh.at[didx_v.at[i * 4 + k]],
                                 sem, add=True)
                for k in range(4)
            ]
            for desc in descs:
                desc.wait()
            return carry

        lax.fori_loop(0, cpt // 4, body, 0)
        plsc.subcore_barrier()
        pltpu.sync_copy(acc_sh.at[pl.ds(sid * rpt, rpt)], stage_v)
        pltpu.sync_copy(stage_v,
                        out_hbm.at[pl.ds(cid * n_pad + sid * rpt, rpt)])

    ones = jnp.ones((_C2,), jnp.float32)
    zeros = jnp.zeros((n_pad,), jnp.float32)
    return deg_kernel(dst2, ones, zeros)


def _sc_propagate(table, src_i32, dst_i32):
    n_nodes, d = table.shape
    e = src_i32.shape[0]
    nw = _NC * _NS
    assert e % (nw * _CHUNK) == 0, e
    per_tile = e // nw
    cpt = per_tile // _CHUNK
    n_pad = _pad_rows(n_nodes)
    rpt = n_pad // _NS
    nb = 4
    groups = cpt // nb
    tail = cpt - nb * groups

    mesh = plsc.VectorSubcoreMesh(core_axis_name="c", subcore_axis_name="s")

    @functools.partial(
        pl.kernel,
        out_type=jax.ShapeDtypeStruct((_NC, n_pad, d), jnp.float32),
        mesh=mesh,
        scratch_types=[
            [pltpu.VMEM((_CHUNK,), jnp.int32)] * 4,
            [pltpu.VMEM((_CHUNK,), jnp.int32)] * 4,
            [pltpu.VMEM((_CHUNK, d), jnp.float32)] * 4,
            pltpu.VMEM_SHARED((n_pad, d), jnp.float32),
            [pltpu.SemaphoreType.DMA] * 4,
        ],
    )
    def prop_kernel(table_hbm, src_hbm, dst_hbm, zeros_h, out_hbm,
                    sidx_v, didx_v, rows_v, acc_sh, semg):
        cid = lax.axis_index("c")
        sid = lax.axis_index("s")
        wid = cid * _NS + sid
        pltpu.sync_copy(zeros_h.at[pl.ds(sid * rpt, rpt)],
                        acc_sh.at[pl.ds(sid * rpt, rpt)])
        plsc.subcore_barrier()
        base0 = wid * per_tile

        def load_and_gather(j, k):
            base = base0 + j * _CHUNK
            pltpu.sync_copy(src_hbm.at[pl.ds(base, _CHUNK)], sidx_v[k])
            pltpu.sync_copy(dst_hbm.at[pl.ds(base, _CHUNK)], didx_v[k])
            return pltpu.async_copy(table_hbm.at[sidx_v[k]],
                                    rows_v[k], semg[k])

        def body(i, carry):
            gs = [load_and_gather(nb * i + k, k) for k in range(nb)]
            for k in range(nb):
                gs[k].wait()
                pltpu.sync_copy(rows_v[k], acc_sh.at[didx_v[k]], add=True)
            return carry

        lax.fori_loop(0, groups, body, 0)
        if tail:
            gs = [load_and_gather(nb * groups + k, k) for k in range(tail)]
            for k in range(tail):
                gs[k].wait()
                pltpu.sync_copy(rows_v[k], acc_sh.at[didx_v[k]], add=True)
        plsc.subcore_barrier()
        pltpu.sync_copy(acc_sh.at[pl.ds(sid * rpt, rpt)],
                        out_hbm.at[cid, pl.ds(sid * rpt, rpt)])

    zeros = jnp.zeros((n_pad, d), jnp.float32)
    return prop_kernel(table, src_i32, dst_i32, zeros)


_ROWS = 1000


def _tc_pre(x, deg0, deg1, w):
    n, d = x.shape
    dout = w.shape[1]
    assert n % _ROWS == 0

    def body(x_ref, d0_ref, d1_ref, w_ref, o_ref):
        dinv = lax.rsqrt(d0_ref[...] + d1_ref[...] + 1.0)
        o_ref[...] = jnp.dot(x_ref[...] * dinv, w_ref[...],
                             preferred_element_type=jnp.float32)

    return pl.pallas_call(
        body,
        grid=(n // _ROWS,),
        in_specs=[
            pl.BlockSpec((_ROWS, d), lambda i: (i, 0)),
            pl.BlockSpec((_ROWS, 1), lambda i: (i, 0)),
            pl.BlockSpec((_ROWS, 1), lambda i: (i, 0)),
            pl.BlockSpec((d, dout), lambda i: (0, 0)),
        ],
        out_specs=pl.BlockSpec((_ROWS, dout), lambda i: (i, 0)),
        out_shape=jax.ShapeDtypeStruct((n, dout), jnp.float32),
    )(x, deg0, deg1, w)


def _tc_mid(p0, p1, z, deg0, deg1, b, w):
    n, d = z.shape
    dout = w.shape[1]

    def body(p0_ref, p1_ref, z_ref, d0_ref, d1_ref, b_ref, w_ref, o_ref):
        dinv = lax.rsqrt(d0_ref[...] + d1_ref[...] + 1.0)
        pre = (p0_ref[...] + p1_ref[...] + z_ref[...]) * dinv + b_ref[...]
        h = jnp.maximum(pre, 0.0) * dinv
        o_ref[...] = jnp.dot(h, w_ref[...], preferred_element_type=jnp.float32)

    return pl.pallas_call(
        body,
        grid=(n // _ROWS,),
        in_specs=[
            pl.BlockSpec((_ROWS, d), lambda i: (i, 0)),
            pl.BlockSpec((_ROWS, d), lambda i: (i, 0)),
            pl.BlockSpec((_ROWS, d), lambda i: (i, 0)),
            pl.BlockSpec((_ROWS, 1), lambda i: (i, 0)),
            pl.BlockSpec((_ROWS, 1), lambda i: (i, 0)),
            pl.BlockSpec((1, d), lambda i: (0, 0)),
            pl.BlockSpec((d, dout), lambda i: (0, 0)),
        ],
        out_specs=pl.BlockSpec((_ROWS, dout), lambda i: (i, 0)),
        out_shape=jax.ShapeDtypeStruct((n, dout), jnp.float32),
    )(p0, p1, z, deg0, deg1, b, w)


def _tc_post(p0, p1, z, deg0, deg1, b, wh, bh):
    n, d = z.shape
    dout = wh.shape[1]

    def body(p0_ref, p1_ref, z_ref, d0_ref, d1_ref, b_ref, wh_ref, bh_ref,
             o_ref):
        dinv = lax.rsqrt(d0_ref[...] + d1_ref[...] + 1.0)
        pre = (p0_ref[...] + p1_ref[...] + z_ref[...]) * dinv + b_ref[...]
        h = jnp.maximum(pre, 0.0)
        o_ref[...] = jnp.dot(h, wh_ref[...],
                             preferred_element_type=jnp.float32) + bh_ref[...]

    return pl.pallas_call(
        body,
        grid=(n // _ROWS,),
        in_specs=[
            pl.BlockSpec((_ROWS, d), lambda i: (i, 0)),
            pl.BlockSpec((_ROWS, d), lambda i: (i, 0)),
            pl.BlockSpec((_ROWS, d), lambda i: (i, 0)),
            pl.BlockSpec((_ROWS, 1), lambda i: (i, 0)),
            pl.BlockSpec((_ROWS, 1), lambda i: (i, 0)),
            pl.BlockSpec((1, d), lambda i: (0, 0)),
            pl.BlockSpec((d, dout), lambda i: (0, 0)),
            pl.BlockSpec((1, dout), lambda i: (0, 0)),
        ],
        out_specs=pl.BlockSpec((_ROWS, dout), lambda i: (i, 0)),
        out_shape=jax.ShapeDtypeStruct((n, dout), jnp.float32),
    )(p0, p1, z, deg0, deg1, b, wh, bh)


def kernel(x, edge_index, W1, b1, W2, b2, Wh, bh):
    n, d = x.shape
    src = edge_index[0].astype(jnp.int32)
    dst = edge_index[1].astype(jnp.int32)
    dst2, cpt = _pad_edges(dst, n)

    n_pad = _pad_rows(n)
    deg2 = _sc_degree(dst2, cpt, n).reshape(_NC, n_pad, 1)
    deg_a, deg_b = deg2[0], deg2[1]

    z1 = _tc_pre(x, deg_a, deg_b, W1)
    pp = _sc_propagate(z1, src, dst)
    z2 = _tc_mid(pp[0], pp[1], z1, deg_a, deg_b, b1.reshape(1, -1), W2)
    qq = _sc_propagate(z2, src, dst)
    out = _tc_post(qq[0], qq[1], z2, deg_a, deg_b, b2.reshape(1, -1),
                   Wh, bh.reshape(1, -1))
    return out
